# baseline (speedup 1.0000x reference)
import functools

import jax
import jax.numpy as jnp
import numpy as np
from jax import lax
from jax.experimental import pallas as pl
from jax.experimental.pallas import tpu as pltpu
from jax.experimental.pallas import tpu_sc as plsc

_BATCH = 16384
_DIM = 512
_NUM_INTERVENTIONS = 128
_LANES = 16


_INTERVENTION_IDX = np.array([
    2, 4, 5, 7, 16, 19, 29, 30, 31, 34, 35, 37, 42, 44, 45, 58, 61, 63,
    65, 72, 78, 82, 83, 85, 90, 99, 101, 102, 108, 110, 111, 112, 114,
    117, 121, 123, 129, 130, 139, 142, 144, 148, 152, 153, 155, 156, 157,
    163, 167, 174, 175, 176, 177, 178, 179, 183, 186, 188, 189, 197, 211,
    212, 219, 240, 251, 254, 257, 259, 263, 268, 269, 272, 275, 277, 278,
    284, 291, 300, 302, 304, 305, 309, 312, 315, 318, 323, 325, 336, 339,
    342, 350, 354, 356, 363, 366, 367, 368, 369, 379, 388, 398, 406, 409,
    410, 415, 417, 429, 436, 441, 444, 446, 447, 448, 452, 461, 462, 463,
    480, 481, 487, 493, 495, 499, 501, 504, 507, 509, 510,
], dtype=np.int32)

_MASK = np.zeros((_DIM,), np.float32)
_MASK[_INTERVENTION_IDX] = 1.0

_NUM_CORES = 2
_NUM_SUBCORES = 16
_NUM_WORKERS = _NUM_CORES * _NUM_SUBCORES
_ROWS_PER_WORKER = _BATCH // _NUM_WORKERS
_CHUNK_ROWS = 32
_NUM_CHUNKS = _ROWS_PER_WORKER // _CHUNK_ROWS
_VECS_PER_ROW = _DIM // _LANES


def _sc_body(x_hbm, c_hbm, m_hbm, out_hbm, xb, cb, mb):
    wid = lax.axis_index("s") * _NUM_CORES + lax.axis_index("c")
    base = wid * _ROWS_PER_WORKER
    pltpu.sync_copy(m_hbm, mb)

    def row_body(r, carry):
        for j in range(_VECS_PER_ROW):
            sl = pl.ds(j * _LANES, _LANES)
            xv = xb[r, sl]
            cv = cb[r, sl]
            mv = mb[sl]
            xb[r, sl] = xv + mv * ((1.0 - cv) - xv)
        return carry

    def chunk_body(k, carry):
        rows = pl.ds(base + k * _CHUNK_ROWS, _CHUNK_ROWS)
        pltpu.sync_copy(x_hbm.at[rows], xb)
        pltpu.sync_copy(c_hbm.at[rows], cb)
        lax.fori_loop(0, _CHUNK_ROWS, row_body, 0)
        pltpu.sync_copy(xb, out_hbm.at[rows])
        return carry

    lax.fori_loop(0, _NUM_CHUNKS, chunk_body, 0)


@functools.partial(jax.jit, static_argnames=())
def _negative_intervention_sc(x, concepts, mask):
    mesh = plsc.VectorSubcoreMesh(core_axis_name="c", subcore_axis_name="s")
    return pl.kernel(
        _sc_body,
        mesh=mesh,
        out_type=jax.ShapeDtypeStruct((_BATCH, _DIM), jnp.float32),
        scratch_types=[
            pltpu.VMEM((_CHUNK_ROWS, _DIM), jnp.float32),
            pltpu.VMEM((_CHUNK_ROWS, _DIM), jnp.float32),
            pltpu.VMEM((_DIM,), jnp.float32),
        ],
    )(x, concepts, mask)


def kernel(x, concepts):
    mask = jnp.asarray(_MASK)
    return _negative_intervention_sc(x, concepts, mask)


# speedup vs baseline: 1.2652x; 1.2652x over previous
import functools

import jax
import jax.numpy as jnp
import numpy as np
from jax import lax
from jax.experimental import pallas as pl
from jax.experimental.pallas import tpu as pltpu
from jax.experimental.pallas import tpu_sc as plsc

_BATCH = 16384
_DIM = 512
_NUM_INTERVENTIONS = 128
_LANES = 16


_INTERVENTION_IDX = np.array([
    2, 4, 5, 7, 16, 19, 29, 30, 31, 34, 35, 37, 42, 44, 45, 58, 61, 63,
    65, 72, 78, 82, 83, 85, 90, 99, 101, 102, 108, 110, 111, 112, 114,
    117, 121, 123, 129, 130, 139, 142, 144, 148, 152, 153, 155, 156, 157,
    163, 167, 174, 175, 176, 177, 178, 179, 183, 186, 188, 189, 197, 211,
    212, 219, 240, 251, 254, 257, 259, 263, 268, 269, 272, 275, 277, 278,
    284, 291, 300, 302, 304, 305, 309, 312, 315, 318, 323, 325, 336, 339,
    342, 350, 354, 356, 363, 366, 367, 368, 369, 379, 388, 398, 406, 409,
    410, 415, 417, 429, 436, 441, 444, 446, 447, 448, 452, 461, 462, 463,
    480, 481, 487, 493, 495, 499, 501, 504, 507, 509, 510,
], dtype=np.int32)

_MASK = np.zeros((_DIM,), np.float32)
_MASK[_INTERVENTION_IDX] = 1.0

_NUM_CORES = 2
_NUM_SUBCORES = 16
_NUM_WORKERS = _NUM_CORES * _NUM_SUBCORES
_ROWS_PER_WORKER = _BATCH // _NUM_WORKERS
_CHUNK_ROWS = 32
_NUM_CHUNKS = _ROWS_PER_WORKER // _CHUNK_ROWS
_VECS_PER_ROW = _DIM // _LANES


def _sc_body(
    x_hbm, c_hbm, m_hbm, out_hbm,
    xb0, xb1, cb0, cb1, ob0, ob1, mb,
    sx0, sx1, sc0, sc1, so0, so1,
):
    wid = lax.axis_index("s") * _NUM_CORES + lax.axis_index("c")
    base = wid * _ROWS_PER_WORKER
    xb = (xb0, xb1)
    cb = (cb0, cb1)
    ob = (ob0, ob1)
    sx = (sx0, sx1)
    sc = (sc0, sc1)
    so = (so0, so1)

    def rows_of(k):
        return pl.ds(base + k * _CHUNK_ROWS, _CHUNK_ROWS)

    pltpu.sync_copy(m_hbm, mb)

    def make_row_body(p):
        def row_body(r, carry):
            for j in range(_VECS_PER_ROW):
                sl = pl.ds(j * _LANES, _LANES)
                xv = xb[p][r, sl]
                cv = cb[p][r, sl]
                mv = mb[sl]
                ob[p][r, sl] = xv + mv * ((1.0 - cv) - xv)
            return carry

        return row_body

    for p in range(2):
        pltpu.async_copy(x_hbm.at[rows_of(p)], xb[p], sx[p])
        pltpu.async_copy(c_hbm.at[rows_of(p)], cb[p], sc[p])

    def step(t, carry):
        for p in range(2):
            k = 2 * t + p
            pltpu.make_async_copy(x_hbm.at[rows_of(k)], xb[p], sx[p]).wait()
            pltpu.make_async_copy(c_hbm.at[rows_of(k)], cb[p], sc[p]).wait()
            lax.fori_loop(0, _CHUNK_ROWS, make_row_body(p), 0)
            pltpu.async_copy(ob[p], out_hbm.at[rows_of(k)], so[p])

            @pl.when(t < _NUM_CHUNKS // 2 - 1)
            def _():
                pltpu.async_copy(x_hbm.at[rows_of(k + 2)], xb[p], sx[p])
                pltpu.async_copy(c_hbm.at[rows_of(k + 2)], cb[p], sc[p])

            pltpu.make_async_copy(ob[p], out_hbm.at[rows_of(k)], so[p]).wait()
        return carry

    lax.fori_loop(0, _NUM_CHUNKS // 2, step, 0)


@functools.partial(jax.jit, static_argnames=())
def _negative_intervention_sc(x, concepts, mask):
    mesh = plsc.VectorSubcoreMesh(core_axis_name="c", subcore_axis_name="s")
    return pl.kernel(
        _sc_body,
        mesh=mesh,
        out_type=jax.ShapeDtypeStruct((_BATCH, _DIM), jnp.float32),
        scratch_types=(
            [pltpu.VMEM((_CHUNK_ROWS, _DIM), jnp.float32)] * 6
            + [pltpu.VMEM((_DIM,), jnp.float32)]
            + [pltpu.SemaphoreType.DMA] * 6
        ),
    )(x, concepts, mask)


def kernel(x, concepts):
    mask = jnp.asarray(_MASK)
    return _negative_intervention_sc(x, concepts, mask)


# speedup vs baseline: 2.7465x; 2.1707x over previous
import functools

import jax
import jax.numpy as jnp
import numpy as np
from jax import lax
from jax.experimental import pallas as pl
from jax.experimental.pallas import tpu as pltpu
from jax.experimental.pallas import tpu_sc as plsc

_BATCH = 16384
_DIM = 512
_NUM_INTERVENTIONS = 128
_LANES = 16


_INTERVENTION_IDX = np.array([
    2, 4, 5, 7, 16, 19, 29, 30, 31, 34, 35, 37, 42, 44, 45, 58, 61, 63,
    65, 72, 78, 82, 83, 85, 90, 99, 101, 102, 108, 110, 111, 112, 114,
    117, 121, 123, 129, 130, 139, 142, 144, 148, 152, 153, 155, 156, 157,
    163, 167, 174, 175, 176, 177, 178, 179, 183, 186, 188, 189, 197, 211,
    212, 219, 240, 251, 254, 257, 259, 263, 268, 269, 272, 275, 277, 278,
    284, 291, 300, 302, 304, 305, 309, 312, 315, 318, 323, 325, 336, 339,
    342, 350, 354, 356, 363, 366, 367, 368, 369, 379, 388, 398, 406, 409,
    410, 415, 417, 429, 436, 441, 444, 446, 447, 448, 452, 461, 462, 463,
    480, 481, 487, 493, 495, 499, 501, 504, 507, 509, 510,
], dtype=np.int32)

_MASK = np.zeros((_DIM,), np.float32)
_MASK[_INTERVENTION_IDX] = 1.0

_NUM_CORES = 2
_NUM_SUBCORES = 16
_NUM_WORKERS = _NUM_CORES * _NUM_SUBCORES
_ROWS_PER_WORKER = _BATCH // _NUM_WORKERS
_CHUNK_ROWS = 32
_NUM_CHUNKS = _ROWS_PER_WORKER // _CHUNK_ROWS
_VECS_PER_ROW = _DIM // _LANES


def _sc_body(
    x_hbm, c_hbm, m_hbm, out_hbm,
    xb0, xb1, cb0, cb1, ob0, ob1, mb,
    sx0, sx1, sc0, sc1, so0, so1,
):
    wid = lax.axis_index("s") * _NUM_CORES + lax.axis_index("c")
    base = wid * _ROWS_PER_WORKER
    xb = (xb0, xb1)
    cb = (cb0, cb1)
    ob = (ob0, ob1)
    sx = (sx0, sx1)
    sc = (sc0, sc1)
    so = (so0, so1)

    def rows_of(k):
        return pl.ds(base + k * _CHUNK_ROWS, _CHUNK_ROWS)

    pltpu.sync_copy(m_hbm, mb)

    def compute_chunk(p):
        for j in range(_VECS_PER_ROW):
            sl = pl.ds(j * _LANES, _LANES)
            mv = mb[sl] > 0.0

            @plsc.parallel_loop(0, _CHUNK_ROWS, unroll=4)
            def _(r):
                xv = xb[p][r, sl]
                cv = cb[p][r, sl]
                ob[p][r, sl] = jnp.where(mv, 1.0 - cv, xv)

    for p in range(2):
        pltpu.async_copy(x_hbm.at[rows_of(p)], xb[p], sx[p])
        pltpu.async_copy(c_hbm.at[rows_of(p)], cb[p], sc[p])

    def step(t, carry):
        for p in range(2):
            k = 2 * t + p
            pltpu.make_async_copy(x_hbm.at[rows_of(k)], xb[p], sx[p]).wait()
            pltpu.make_async_copy(c_hbm.at[rows_of(k)], cb[p], sc[p]).wait()

            @pl.when(t > 0)
            def _():
                pltpu.make_async_copy(
                    ob[p], out_hbm.at[rows_of(k - 2)], so[p]
                ).wait()

            compute_chunk(p)
            pltpu.async_copy(ob[p], out_hbm.at[rows_of(k)], so[p])

            @pl.when(t < _NUM_CHUNKS // 2 - 1)
            def _():
                pltpu.async_copy(x_hbm.at[rows_of(k + 2)], xb[p], sx[p])
                pltpu.async_copy(c_hbm.at[rows_of(k + 2)], cb[p], sc[p])
        return carry

    lax.fori_loop(0, _NUM_CHUNKS // 2, step, 0)

    for p in range(2):
        pltpu.make_async_copy(
            ob[p], out_hbm.at[rows_of(_NUM_CHUNKS - 2 + p)], so[p]
        ).wait()


@functools.partial(jax.jit, static_argnames=())
def _negative_intervention_sc(x, concepts, mask):
    mesh = plsc.VectorSubcoreMesh(core_axis_name="c", subcore_axis_name="s")
    return pl.kernel(
        _sc_body,
        mesh=mesh,
        out_type=jax.ShapeDtypeStruct((_BATCH, _DIM), jnp.float32),
        scratch_types=(
            [pltpu.VMEM((_CHUNK_ROWS, _DIM), jnp.float32)] * 6
            + [pltpu.VMEM((_DIM,), jnp.float32)]
            + [pltpu.SemaphoreType.DMA] * 6
        ),
    )(x, concepts, mask)


def kernel(x, concepts):
    mask = jnp.asarray(_MASK)
    return _negative_intervention_sc(x, concepts, mask)
